# Initial kernel scaffold; baseline (speedup 1.0000x reference)
import jax
import jax.numpy as jnp
from jax.experimental import pallas as pl


def kernel(x, edge_index, edge_attr, W_lin, b_lin, W_edge, b_edge, root_emb):
    raise NotImplementedError("write your pallas kernel here")


# speedup vs baseline: 3.2187x; 3.2187x over previous
import jax
import jax.numpy as jnp
from jax import lax
from jax.experimental import pallas as pl
from jax.experimental.pallas import tpu as pltpu
from jax.experimental.pallas import tpu_sc as plsc

N = 10000
E = 320000
D = 128
DH = D // 2

NC = 2
NS = 16
NW = NC * NS

EPW = E // NW
CH = 80
NP = 10240
RPS = NP // NS
FL = 128


def _mesh():
    return plsc.VectorSubcoreMesh(core_axis_name="c", subcore_axis_name="s")


def _deg_body(row_hbm, parts_hbm, row_v, deg_v):
    c = lax.axis_index("c")
    s = lax.axis_index("s")
    w = s * NC + c

    def zero(i, _):
        deg_v[pl.ds(i * 16, 16)] = jnp.zeros((16,), jnp.float32)
        return 0

    lax.fori_loop(0, N // 16, zero, 0)

    pltpu.sync_copy(row_hbm.at[pl.ds(w * EPW, EPW)], row_v)

    ones = jnp.ones((16,), jnp.float32)

    def hist(g, _):
        idx16 = row_v[pl.ds(g * 16, 16)]
        plsc.addupdate_scatter(deg_v, [idx16], ones)
        return 0

    lax.fori_loop(0, EPW // 16, hist, 0)

    pltpu.sync_copy(deg_v, parts_hbm.at[pl.ds(w * N, N)])


def _deg(row):
    return pl.kernel(
        _deg_body,
        out_type=jax.ShapeDtypeStruct((NW * N,), jnp.float32),
        mesh=_mesh(),
        scratch_types=[
            pltpu.VMEM((EPW,), jnp.int32),
            pltpu.VMEM((N,), jnp.float32),
        ],
        compiler_params=pltpu.CompilerParams(needs_layout_passes=False),
    )(row)


def _main_body(h0_hbm, h1_hbm, e0_hbm, e1_hbm, row_hbm, col_hbm, dinv_hbm,
               parts_hbm, dinv_v, ri_v, sc_v, row_v, col_v, hr_v, e_v, m_v,
               fl_v, aggr_sh, sem):
    c = lax.axis_index("c")
    s = lax.axis_index("s")
    base0 = (c * NS + s) * EPW

    def zero(i, _):
        fl_v[i // 4, pl.ds((i % 4) * 16, 16)] = jnp.zeros((16,), jnp.float32)
        return 0

    lax.fori_loop(0, FL * (DH // 16), zero, 0)

    pltpu.sync_copy(dinv_hbm, dinv_v)
    pltpu.sync_copy(row_hbm.at[pl.ds(base0, EPW)], ri_v)

    def scale(g, _):
        rv = ri_v[pl.ds(g * 16, 16)]
        sc_v[pl.ds(g * 16, 16)] = plsc.load_gather(dinv_v, [rv])
        return 0

    lax.fori_loop(0, EPW // 16, scale, 0)

    for half in range(2):
        h_hbm = h0_hbm if half == 0 else h1_hbm
        e_hbm = e0_hbm if half == 0 else e1_hbm

        for t in range(RPS // FL):
            pltpu.sync_copy(fl_v, aggr_sh.at[pl.ds(s * RPS + t * FL, FL)])
        plsc.subcore_barrier()

        def chunk(i, _):
            base = base0 + i * CH
            pltpu.sync_copy(col_hbm.at[pl.ds(base, CH)], col_v)
            pltpu.sync_copy(row_hbm.at[pl.ds(base, CH)], row_v)
            pltpu.async_copy(h_hbm.at[row_v], hr_v, sem).wait()
            pltpu.sync_copy(e_hbm.at[pl.ds(base, CH)], e_v)

            def group(g, _):
                dr = sc_v[pl.ds(i * CH + g * 16, 16)]
                for jj in range(16):
                    j = g * 16 + jj
                    nj = dr[jj]
                    for d in range(DH // 16):
                        v = (hr_v[j, pl.ds(d * 16, 16)]
                             + e_v[j, pl.ds(d * 16, 16)])
                        m_v[j, pl.ds(d * 16, 16)] = jnp.maximum(v, 0.0) * nj
                return 0

            lax.fori_loop(0, CH // 16, group, 0)

            pltpu.sync_copy(m_v, aggr_sh.at[col_v], add=True)
            return 0

        lax.fori_loop(0, EPW // CH, chunk, 0)
        plsc.subcore_barrier()

        for t in range(RPS // FL):
            r0 = s * RPS + t * FL
            pltpu.sync_copy(aggr_sh.at[pl.ds(r0, FL)], fl_v)
            pltpu.sync_copy(fl_v, parts_hbm.at[half, c, pl.ds(r0, FL)])

        if half == 0:
            lax.fori_loop(0, FL * (DH // 16), zero, 0)


def _main(h0, h1, e0, e1, row, col, dinv):
    return pl.kernel(
        _main_body,
        out_type=jax.ShapeDtypeStruct((2, NC, NP, DH), jnp.float32),
        mesh=_mesh(),
        scratch_types=[
            pltpu.VMEM((N,), jnp.float32),
            pltpu.VMEM((EPW,), jnp.int32),
            pltpu.VMEM((EPW,), jnp.float32),
            pltpu.VMEM((CH,), jnp.int32),
            pltpu.VMEM((CH,), jnp.int32),
            pltpu.VMEM((CH, DH), jnp.float32),
            pltpu.VMEM((CH, DH), jnp.float32),
            pltpu.VMEM((CH, DH), jnp.float32),
            pltpu.VMEM((FL, DH), jnp.float32),
            pltpu.VMEM_SHARED((NP, DH), jnp.float32),
            pltpu.SemaphoreType.DMA,
        ],
        compiler_params=pltpu.CompilerParams(
            needs_layout_passes=False, use_tc_tiling_on_sc=False),
    )(h0, h1, e0, e1, row, col, dinv)


def _mm_body(x_ref, w_ref, b_ref, o_ref):
    o_ref[...] = lax.dot_general(
        x_ref[...], w_ref[...], (((1,), (1,)), ((), ())),
        preferred_element_type=jnp.float32) + b_ref[...]


def _linear(x, w, b, blk):
    n = x.shape[0]
    dout = w.shape[0]
    return pl.pallas_call(
        _mm_body,
        grid=(n // blk,),
        in_specs=[
            pl.BlockSpec((blk, x.shape[1]), lambda i: (i, 0)),
            pl.BlockSpec(w.shape, lambda i: (0, 0)),
            pl.BlockSpec((1, dout), lambda i: (0, 0)),
        ],
        out_specs=pl.BlockSpec((blk, dout), lambda i: (i, 0)),
        out_shape=jax.ShapeDtypeStruct((n, dout), jnp.float32),
    )(x, w, b.reshape(1, dout))


def _final_body(p_ref, h_ref, root_ref, recip_ref, dinv_ref, o_ref):
    ob = jnp.maximum(h_ref[...] + root_ref[...], 0.0) * recip_ref[...]
    aggr = jnp.concatenate(
        [p_ref[0, 0] + p_ref[0, 1], p_ref[1, 0] + p_ref[1, 1]], axis=-1)
    o_ref[...] = aggr * dinv_ref[...] + ob


def _final(parts, h, root_emb, recip, dinv):
    blk = 400
    return pl.pallas_call(
        _final_body,
        grid=(N // blk,),
        in_specs=[
            pl.BlockSpec((2, NC, blk, DH), lambda i: (0, 0, i, 0)),
            pl.BlockSpec((blk, D), lambda i: (i, 0)),
            pl.BlockSpec((1, D), lambda i: (0, 0)),
            pl.BlockSpec((blk, 1), lambda i: (i, 0)),
            pl.BlockSpec((blk, 1), lambda i: (i, 0)),
        ],
        out_specs=pl.BlockSpec((blk, D), lambda i: (i, 0)),
        out_shape=jax.ShapeDtypeStruct((N, D), jnp.float32),
    )(parts, h, root_emb, recip.reshape(N, 1), dinv.reshape(N, 1))


@jax.jit
def kernel(x, edge_index, edge_attr, W_lin, b_lin, W_edge, b_edge, root_emb):
    row = edge_index[0]
    col = edge_index[1]

    deg_parts = _deg(row)
    deg = jnp.sum(deg_parts.reshape(NW, N), axis=0) + 1.0
    dinv = deg ** -0.5
    recip = 1.0 / deg

    h = _linear(x, W_lin, b_lin, 400)
    h0 = h[:, :DH]
    h1 = h[:, DH:]
    e0 = _linear(edge_attr, W_edge[:DH], b_edge[:DH], 640)
    e1 = _linear(edge_attr, W_edge[DH:], b_edge[DH:], 640)

    parts = _main(h0, h1, e0, e1, row, col, dinv)
    return _final(parts, h, root_emb, recip, dinv)


# speedup vs baseline: 5.0204x; 1.5598x over previous
import jax
import jax.numpy as jnp
from jax import lax
from jax.experimental import pallas as pl
from jax.experimental.pallas import tpu as pltpu
from jax.experimental.pallas import tpu_sc as plsc

N = 10000
E = 320000
D = 128

NC = 2
NS = 16
NW = NC * NS

HALF = N // NC
EPT = E // NS
CH = 80
NCH = EPT // CH
NPB = 5632
TRASH = 5500
RPS = NPB // NS
FL = 176

EPW = E // NW


def _mesh():
    return plsc.VectorSubcoreMesh(core_axis_name="c", subcore_axis_name="s")


def _deg_body(row_hbm, parts_hbm, row_v, deg_v):
    c = lax.axis_index("c")
    s = lax.axis_index("s")
    w = s * NC + c

    def zero(i, _):
        deg_v[pl.ds(i * 16, 16)] = jnp.zeros((16,), jnp.float32)
        return 0

    lax.fori_loop(0, N // 16, zero, 0)

    pltpu.sync_copy(row_hbm.at[pl.ds(w * EPW, EPW)], row_v)

    ones = jnp.ones((16,), jnp.float32)

    def hist(g, _):
        idx16 = row_v[pl.ds(g * 16, 16)]
        plsc.addupdate_scatter(deg_v, [idx16], ones)
        return 0

    lax.fori_loop(0, EPW // 16, hist, 0)

    pltpu.sync_copy(deg_v, parts_hbm.at[pl.ds(w * N, N)])


def _deg(row):
    return pl.kernel(
        _deg_body,
        out_type=jax.ShapeDtypeStruct((NW * N,), jnp.float32),
        mesh=_mesh(),
        scratch_types=[
            pltpu.VMEM((EPW,), jnp.int32),
            pltpu.VMEM((N,), jnp.float32),
        ],
        compiler_params=pltpu.CompilerParams(needs_layout_passes=False),
    )(row)


def _main_body(h_hbm, e_hbm, row_hbm, col_hbm, dinv_hbm, parts_hbm,
               dinv_v, row_v, col_v, loc_v, hr_v, e_v, m_v, fl_v,
               aggr_sh, sem):
    c = lax.axis_index("c")
    s = lax.axis_index("s")
    base0 = s * EPT
    lo = c * HALF

    def zero(i, _):
        fl_v[i // 8, pl.ds((i % 8) * 16, 16)] = jnp.zeros((16,), jnp.float32)
        return 0

    lax.fori_loop(0, FL * (D // 16), zero, 0)
    for t in range(RPS // FL):
        pltpu.sync_copy(fl_v, aggr_sh.at[pl.ds(s * RPS + t * FL, FL)])

    pltpu.sync_copy(dinv_hbm, dinv_v)
    plsc.subcore_barrier()

    def chunk(i, _):
        base = base0 + i * CH
        pltpu.sync_copy(col_hbm.at[pl.ds(base, CH)], col_v)
        pltpu.sync_copy(row_hbm.at[pl.ds(base, CH)], row_v)
        pltpu.async_copy(h_hbm.at[row_v], hr_v, sem).wait()
        pltpu.sync_copy(e_hbm.at[pl.ds(base, CH)], e_v)

        def group(g, _):
            rv = row_v[pl.ds(g * 16, 16)]
            dr = plsc.load_gather(dinv_v, [rv])
            cv = col_v[pl.ds(g * 16, 16)]
            lc = cv - lo
            ok = (lc >= 0) & (lc < HALF)
            loc_v[pl.ds(g * 16, 16)] = jnp.where(ok, lc, TRASH)
            for jj in range(16):
                j = g * 16 + jj
                nj = dr[jj]
                for d in range(D // 16):
                    v = (hr_v[j, pl.ds(d * 16, 16)]
                         + e_v[j, pl.ds(d * 16, 16)])
                    m_v[j, pl.ds(d * 16, 16)] = jnp.maximum(v, 0.0) * nj
            return 0

        lax.fori_loop(0, CH // 16, group, 0)

        pltpu.sync_copy(m_v, aggr_sh.at[loc_v], add=True)
        return 0

    lax.fori_loop(0, NCH, chunk, 0)
    plsc.subcore_barrier()

    for t in range(RPS // FL):
        r0 = s * RPS + t * FL
        pltpu.sync_copy(aggr_sh.at[pl.ds(r0, FL)], fl_v)
        pltpu.sync_copy(fl_v, parts_hbm.at[c, pl.ds(r0, FL)])


def _main(h, e, row, col, dinv):
    return pl.kernel(
        _main_body,
        out_type=jax.ShapeDtypeStruct((NC, NPB, D), jnp.float32),
        mesh=_mesh(),
        scratch_types=[
            pltpu.VMEM((N,), jnp.float32),
            pltpu.VMEM((CH,), jnp.int32),
            pltpu.VMEM((CH,), jnp.int32),
            pltpu.VMEM((CH,), jnp.int32),
            pltpu.VMEM((CH, D), jnp.float32),
            pltpu.VMEM((CH, D), jnp.float32),
            pltpu.VMEM((CH, D), jnp.float32),
            pltpu.VMEM((FL, D), jnp.float32),
            pltpu.VMEM_SHARED((NPB, D), jnp.float32),
            pltpu.SemaphoreType.DMA,
        ],
        compiler_params=pltpu.CompilerParams(needs_layout_passes=False),
    )(h, e, row, col, dinv)


def _mm_body(x_ref, w_ref, b_ref, o_ref):
    o_ref[...] = lax.dot_general(
        x_ref[...], w_ref[...], (((1,), (1,)), ((), ())),
        preferred_element_type=jnp.float32) + b_ref[...]


def _linear(x, w, b, blk):
    n = x.shape[0]
    dout = w.shape[0]
    return pl.pallas_call(
        _mm_body,
        grid=(n // blk,),
        in_specs=[
            pl.BlockSpec((blk, x.shape[1]), lambda i: (i, 0)),
            pl.BlockSpec(w.shape, lambda i: (0, 0)),
            pl.BlockSpec((1, dout), lambda i: (0, 0)),
        ],
        out_specs=pl.BlockSpec((blk, dout), lambda i: (i, 0)),
        out_shape=jax.ShapeDtypeStruct((n, dout), jnp.float32),
    )(x, w, b.reshape(1, dout))


def _final_body(p_ref, h_ref, root_ref, recip_ref, dinv_ref, o_ref):
    ob = jnp.maximum(h_ref[...] + root_ref[...], 0.0) * recip_ref[...]
    o_ref[...] = p_ref[...] * dinv_ref[...] + ob


def _final(aggr, h, root_emb, recip, dinv):
    blk = 400
    return pl.pallas_call(
        _final_body,
        grid=(N // blk,),
        in_specs=[
            pl.BlockSpec((blk, D), lambda i: (i, 0)),
            pl.BlockSpec((blk, D), lambda i: (i, 0)),
            pl.BlockSpec((1, D), lambda i: (0, 0)),
            pl.BlockSpec((blk, 1), lambda i: (i, 0)),
            pl.BlockSpec((blk, 1), lambda i: (i, 0)),
        ],
        out_specs=pl.BlockSpec((blk, D), lambda i: (i, 0)),
        out_shape=jax.ShapeDtypeStruct((N, D), jnp.float32),
    )(aggr, h, root_emb, recip.reshape(N, 1), dinv.reshape(N, 1))


@jax.jit
def kernel(x, edge_index, edge_attr, W_lin, b_lin, W_edge, b_edge, root_emb):
    row = edge_index[0]
    col = edge_index[1]

    deg_parts = _deg(row)
    deg = jnp.sum(deg_parts.reshape(NW, N), axis=0) + 1.0
    dinv = deg ** -0.5
    recip = 1.0 / deg

    h = _linear(x, W_lin, b_lin, 400)
    e = _linear(edge_attr, W_edge, b_edge, 3200)

    parts = _main(h, e, row, col, dinv)
    aggr = jnp.concatenate([parts[0, :HALF], parts[1, :HALF]], axis=0)
    return _final(aggr, h, root_emb, recip, dinv)


# speedup vs baseline: 7.7342x; 1.5406x over previous
import jax
import jax.numpy as jnp
from jax import lax
from jax.experimental import pallas as pl
from jax.experimental.pallas import tpu as pltpu
from jax.experimental.pallas import tpu_sc as plsc

N = 10000
E = 320000
D = 128

NC = 2
NS = 16
NW = NC * NS

HALF = N // NC
EPT = E // NS
CH = 80
NCH = EPT // CH
NPB = 5632
TRASH = 5500
RPS = NPB // NS
FL = 176

EPW = E // NW


def _mesh():
    return plsc.VectorSubcoreMesh(core_axis_name="c", subcore_axis_name="s")


def _deg_body(row_hbm, parts_hbm, row_v, deg_v):
    c = lax.axis_index("c")
    s = lax.axis_index("s")
    w = s * NC + c

    def zero(i, _):
        deg_v[pl.ds(i * 16, 16)] = jnp.zeros((16,), jnp.float32)
        return 0

    lax.fori_loop(0, N // 16, zero, 0)

    pltpu.sync_copy(row_hbm.at[pl.ds(w * EPW, EPW)], row_v)

    ones = jnp.ones((16,), jnp.float32)

    def hist(g, _):
        idx16 = row_v[pl.ds(g * 16, 16)]
        plsc.addupdate_scatter(deg_v, [idx16], ones)
        return 0

    lax.fori_loop(0, EPW // 16, hist, 0)

    pltpu.sync_copy(deg_v, parts_hbm.at[pl.ds(w * N, N)])


def _deg(row):
    return pl.kernel(
        _deg_body,
        out_type=jax.ShapeDtypeStruct((NW * N,), jnp.float32),
        mesh=_mesh(),
        scratch_types=[
            pltpu.VMEM((EPW,), jnp.int32),
            pltpu.VMEM((N,), jnp.float32),
        ],
        compiler_params=pltpu.CompilerParams(needs_layout_passes=False),
    )(row)


def _main_body(h_hbm, e_hbm, row_hbm, col_hbm, dinv_hbm, parts_hbm,
               dinv_v, row0_v, row1_v, col0_v, col1_v, loc_v,
               hr_v, e0_v, e1_v, m_v, fl_v,
               aggr_sh, gsem, srow, scol, se):
    c = lax.axis_index("c")
    s = lax.axis_index("s")
    base0 = s * EPT
    lo = c * HALF
    row_b = (row0_v, row1_v)
    col_b = (col0_v, col1_v)
    e_b = (e0_v, e1_v)

    def zero(i, _):
        fl_v[i // 8, pl.ds((i % 8) * 16, 16)] = jnp.zeros((16,), jnp.float32)
        return 0

    lax.fori_loop(0, FL * (D // 16), zero, 0)
    for t in range(RPS // FL):
        pltpu.sync_copy(fl_v, aggr_sh.at[pl.ds(s * RPS + t * FL, FL)])

    pltpu.sync_copy(dinv_hbm, dinv_v)
    plsc.subcore_barrier()

    def pre_start(i, b):
        base = base0 + i * CH
        pltpu.async_copy(row_hbm.at[pl.ds(base, CH)], row_b[b], srow[b])
        pltpu.async_copy(col_hbm.at[pl.ds(base, CH)], col_b[b], scol[b])
        pltpu.async_copy(e_hbm.at[pl.ds(base, CH)], e_b[b], se[b])

    def pre_wait(b):
        pltpu.make_async_copy(row_hbm.at[pl.ds(0, CH)], row_b[b],
                              srow[b]).wait()
        pltpu.make_async_copy(col_hbm.at[pl.ds(0, CH)], col_b[b],
                              scol[b]).wait()
        pltpu.make_async_copy(e_hbm.at[pl.ds(0, CH)], e_b[b], se[b]).wait()

    pre_start(0, 0)
    pre_start(1, 1)

    def pair(k, _):
        for b in range(2):
            i = 2 * k + b
            pre_wait(b)
            pltpu.async_copy(h_hbm.at[row_b[b]], hr_v, gsem).wait()

            def group(g, _):
                rv = row_b[b][pl.ds(g * 16, 16)]
                dr = plsc.load_gather(dinv_v, [rv])
                cv = col_b[b][pl.ds(g * 16, 16)]
                lc = cv - lo
                ok = (lc >= 0) & (lc < HALF)
                loc_v[pl.ds(g * 16, 16)] = jnp.where(ok, lc, TRASH)
                for jj in range(16):
                    j = g * 16 + jj
                    nj = dr[jj]
                    for d in range(D // 16):
                        v = (hr_v[j, pl.ds(d * 16, 16)]
                             + e_b[b][j, pl.ds(d * 16, 16)])
                        m_v[j, pl.ds(d * 16, 16)] = (
                            jnp.maximum(v, 0.0) * nj)
                return 0

            lax.fori_loop(0, CH // 16, group, 0)

            ii = jnp.where(i + 2 < NCH, i + 2, 0)
            pre_start(ii, b)

            pltpu.sync_copy(m_v, aggr_sh.at[loc_v], add=True)
        return 0

    lax.fori_loop(0, NCH // 2, pair, 0)
    pre_wait(0)
    pre_wait(1)
    plsc.subcore_barrier()

    for t in range(RPS // FL):
        r0 = s * RPS + t * FL
        pltpu.sync_copy(aggr_sh.at[pl.ds(r0, FL)], fl_v)
        pltpu.sync_copy(fl_v, parts_hbm.at[c, pl.ds(r0, FL)])


def _main(h, e, row, col, dinv):
    return pl.kernel(
        _main_body,
        out_type=jax.ShapeDtypeStruct((NC, NPB, D), jnp.float32),
        mesh=_mesh(),
        scratch_types=[
            pltpu.VMEM((N,), jnp.float32),
            pltpu.VMEM((CH,), jnp.int32),
            pltpu.VMEM((CH,), jnp.int32),
            pltpu.VMEM((CH,), jnp.int32),
            pltpu.VMEM((CH,), jnp.int32),
            pltpu.VMEM((CH,), jnp.int32),
            pltpu.VMEM((CH, D), jnp.float32),
            pltpu.VMEM((CH, D), jnp.float32),
            pltpu.VMEM((CH, D), jnp.float32),
            pltpu.VMEM((CH, D), jnp.float32),
            pltpu.VMEM((FL, D), jnp.float32),
            pltpu.VMEM_SHARED((NPB, D), jnp.float32),
            pltpu.SemaphoreType.DMA,
            [pltpu.SemaphoreType.DMA, pltpu.SemaphoreType.DMA],
            [pltpu.SemaphoreType.DMA, pltpu.SemaphoreType.DMA],
            [pltpu.SemaphoreType.DMA, pltpu.SemaphoreType.DMA],
        ],        compiler_params=pltpu.CompilerParams(needs_layout_passes=False),
    )(h, e, row, col, dinv)


def _mm_body(x_ref, w_ref, b_ref, o_ref):
    o_ref[...] = lax.dot_general(
        x_ref[...], w_ref[...], (((1,), (1,)), ((), ())),
        preferred_element_type=jnp.float32) + b_ref[...]


def _linear(x, w, b, blk):
    n = x.shape[0]
    dout = w.shape[0]
    return pl.pallas_call(
        _mm_body,
        grid=(n // blk,),
        in_specs=[
            pl.BlockSpec((blk, x.shape[1]), lambda i: (i, 0)),
            pl.BlockSpec(w.shape, lambda i: (0, 0)),
            pl.BlockSpec((1, dout), lambda i: (0, 0)),
        ],
        out_specs=pl.BlockSpec((blk, dout), lambda i: (i, 0)),
        out_shape=jax.ShapeDtypeStruct((n, dout), jnp.float32),
    )(x, w, b.reshape(1, dout))


def _final_body(p_ref, h_ref, root_ref, recip_ref, dinv_ref, o_ref):
    ob = jnp.maximum(h_ref[...] + root_ref[...], 0.0) * recip_ref[...]
    o_ref[...] = p_ref[...] * dinv_ref[...] + ob


def _final(aggr, h, root_emb, recip, dinv):
    blk = 400
    return pl.pallas_call(
        _final_body,
        grid=(N // blk,),
        in_specs=[
            pl.BlockSpec((blk, D), lambda i: (i, 0)),
            pl.BlockSpec((blk, D), lambda i: (i, 0)),
            pl.BlockSpec((1, D), lambda i: (0, 0)),
            pl.BlockSpec((blk, 1), lambda i: (i, 0)),
            pl.BlockSpec((blk, 1), lambda i: (i, 0)),
        ],
        out_specs=pl.BlockSpec((blk, D), lambda i: (i, 0)),
        out_shape=jax.ShapeDtypeStruct((N, D), jnp.float32),
    )(aggr, h, root_emb, recip.reshape(N, 1), dinv.reshape(N, 1))


@jax.jit
def kernel(x, edge_index, edge_attr, W_lin, b_lin, W_edge, b_edge, root_emb):
    row = edge_index[0]
    col = edge_index[1]

    deg_parts = _deg(row)
    deg = jnp.sum(deg_parts.reshape(NW, N), axis=0) + 1.0
    dinv = deg ** -0.5
    recip = 1.0 / deg

    h = _linear(x, W_lin, b_lin, 400)
    e = _linear(edge_attr, W_edge, b_edge, 6400)

    parts = _main(h, e, row, col, dinv)
    aggr = jnp.concatenate([parts[0, :HALF], parts[1, :HALF]], axis=0)
    return _final(aggr, h, root_emb, recip, dinv)


# speedup vs baseline: 7.8669x; 1.0172x over previous
import jax
import jax.numpy as jnp
from jax import lax
from jax.experimental import pallas as pl
from jax.experimental.pallas import tpu as pltpu
from jax.experimental.pallas import tpu_sc as plsc

N = 10000
E = 320000
D = 128

NC = 2
NS = 16
NW = NC * NS

HALF = N // NC
EPT = E // NS
CH = 80
NCH = EPT // CH
NPB = 5632
TRASH = 5500
RPS = NPB // NS
FL = 176

EPW = E // NW


def _mesh():
    return plsc.VectorSubcoreMesh(core_axis_name="c", subcore_axis_name="s")


def _deg_body(row_hbm, parts_hbm, row_v, deg_v):
    c = lax.axis_index("c")
    s = lax.axis_index("s")
    w = s * NC + c

    def zero(i, _):
        deg_v[pl.ds(i * 16, 16)] = jnp.zeros((16,), jnp.float32)
        return 0

    lax.fori_loop(0, N // 16, zero, 0)

    pltpu.sync_copy(row_hbm.at[pl.ds(w * EPW, EPW)], row_v)

    ones = jnp.ones((16,), jnp.float32)

    def hist(g, _):
        idx16 = row_v[pl.ds(g * 16, 16)]
        plsc.addupdate_scatter(deg_v, [idx16], ones)
        return 0

    lax.fori_loop(0, EPW // 16, hist, 0)

    pltpu.sync_copy(deg_v, parts_hbm.at[pl.ds(w * N, N)])


def _deg(row):
    return pl.kernel(
        _deg_body,
        out_type=jax.ShapeDtypeStruct((NW * N,), jnp.float32),
        mesh=_mesh(),
        scratch_types=[
            pltpu.VMEM((EPW,), jnp.int32),
            pltpu.VMEM((N,), jnp.float32),
        ],
        compiler_params=pltpu.CompilerParams(needs_layout_passes=False),
    )(row)


def _main_body(h_hbm, e_hbm, row_hbm, col_hbm, dinv_hbm, parts_hbm,
               dinv_v, row0_v, row1_v, col0_v, col1_v, loc_v,
               hr0_v, hr1_v, e0_v, e1_v, m_v, fl_v,
               aggr_sh, gsem, srow, scol, se):
    c = lax.axis_index("c")
    s = lax.axis_index("s")
    base0 = s * EPT
    lo = c * HALF
    row_b = (row0_v, row1_v)
    col_b = (col0_v, col1_v)
    e_b = (e0_v, e1_v)
    hr_b = (hr0_v, hr1_v)

    def zero(i, _):
        fl_v[i // 8, pl.ds((i % 8) * 16, 16)] = jnp.zeros((16,), jnp.float32)
        return 0

    lax.fori_loop(0, FL * (D // 16), zero, 0)
    for t in range(RPS // FL):
        pltpu.sync_copy(fl_v, aggr_sh.at[pl.ds(s * RPS + t * FL, FL)])

    pltpu.sync_copy(dinv_hbm, dinv_v)
    plsc.subcore_barrier()

    def pre_start(i, b):
        base = base0 + i * CH
        pltpu.async_copy(row_hbm.at[pl.ds(base, CH)], row_b[b], srow[b])
        pltpu.async_copy(col_hbm.at[pl.ds(base, CH)], col_b[b], scol[b])
        pltpu.async_copy(e_hbm.at[pl.ds(base, CH)], e_b[b], se[b])

    def pre_wait(b):
        pltpu.make_async_copy(row_hbm.at[pl.ds(0, CH)], row_b[b],
                              srow[b]).wait()
        pltpu.make_async_copy(col_hbm.at[pl.ds(0, CH)], col_b[b],
                              scol[b]).wait()
        pltpu.make_async_copy(e_hbm.at[pl.ds(0, CH)], e_b[b], se[b]).wait()

    pre_start(0, 0)
    pre_start(1, 1)

    def pair(k, _):
        pre_wait(0)
        pre_wait(1)
        g0 = pltpu.async_copy(h_hbm.at[row0_v], hr0_v, gsem)
        g1 = pltpu.async_copy(h_hbm.at[row1_v], hr1_v, gsem)
        g0.wait()
        g1.wait()
        for b in range(2):
            i = 2 * k + b

            def group(g, _):
                rv = row_b[b][pl.ds(g * 16, 16)]
                dr = plsc.load_gather(dinv_v, [rv])
                cv = col_b[b][pl.ds(g * 16, 16)]
                lc = cv - lo
                ok = (lc >= 0) & (lc < HALF)
                loc_v[pl.ds(g * 16, 16)] = jnp.where(ok, lc, TRASH)
                for jj in range(16):
                    j = g * 16 + jj
                    nj = dr[jj]
                    for d in range(D // 16):
                        v = (hr_b[b][j, pl.ds(d * 16, 16)]
                             + e_b[b][j, pl.ds(d * 16, 16)])
                        m_v[j, pl.ds(d * 16, 16)] = (
                            jnp.maximum(v, 0.0) * nj)
                return 0

            lax.fori_loop(0, CH // 16, group, 0)

            ii = jnp.where(i + 2 < NCH, i + 2, 0)
            pre_start(ii, b)

            pltpu.sync_copy(m_v, aggr_sh.at[loc_v], add=True)
        return 0

    lax.fori_loop(0, NCH // 2, pair, 0)
    pre_wait(0)
    pre_wait(1)
    plsc.subcore_barrier()

    for t in range(RPS // FL):
        r0 = s * RPS + t * FL
        pltpu.sync_copy(aggr_sh.at[pl.ds(r0, FL)], fl_v)
        pltpu.sync_copy(fl_v, parts_hbm.at[c, pl.ds(r0, FL)])


def _main(h, e, row, col, dinv):
    return pl.kernel(
        _main_body,
        out_type=jax.ShapeDtypeStruct((NC, NPB, D), jnp.float32),
        mesh=_mesh(),
        scratch_types=[
            pltpu.VMEM((N,), jnp.float32),
            pltpu.VMEM((CH,), jnp.int32),
            pltpu.VMEM((CH,), jnp.int32),
            pltpu.VMEM((CH,), jnp.int32),
            pltpu.VMEM((CH,), jnp.int32),
            pltpu.VMEM((CH,), jnp.int32),
            pltpu.VMEM((CH, D), jnp.float32),
            pltpu.VMEM((CH, D), jnp.float32),
            pltpu.VMEM((CH, D), jnp.float32),
            pltpu.VMEM((CH, D), jnp.float32),
            pltpu.VMEM((CH, D), jnp.float32),
            pltpu.VMEM((FL, D), jnp.float32),
            pltpu.VMEM_SHARED((NPB, D), jnp.float32),
            pltpu.SemaphoreType.DMA,
            [pltpu.SemaphoreType.DMA, pltpu.SemaphoreType.DMA],
            [pltpu.SemaphoreType.DMA, pltpu.SemaphoreType.DMA],
            [pltpu.SemaphoreType.DMA, pltpu.SemaphoreType.DMA],
        ],        compiler_params=pltpu.CompilerParams(needs_layout_passes=False),
    )(h, e, row, col, dinv)


def _mm_body(x_ref, w_ref, b_ref, o_ref):
    o_ref[...] = lax.dot_general(
        x_ref[...], w_ref[...], (((1,), (1,)), ((), ())),
        preferred_element_type=jnp.float32) + b_ref[...]


def _linear(x, w, b, blk):
    n = x.shape[0]
    dout = w.shape[0]
    return pl.pallas_call(
        _mm_body,
        grid=(n // blk,),
        in_specs=[
            pl.BlockSpec((blk, x.shape[1]), lambda i: (i, 0)),
            pl.BlockSpec(w.shape, lambda i: (0, 0)),
            pl.BlockSpec((1, dout), lambda i: (0, 0)),
        ],
        out_specs=pl.BlockSpec((blk, dout), lambda i: (i, 0)),
        out_shape=jax.ShapeDtypeStruct((n, dout), jnp.float32),
    )(x, w, b.reshape(1, dout))


def _final_body(p_ref, h_ref, root_ref, recip_ref, dinv_ref, o_ref):
    ob = jnp.maximum(h_ref[...] + root_ref[...], 0.0) * recip_ref[...]
    o_ref[...] = p_ref[...] * dinv_ref[...] + ob


def _final(aggr, h, root_emb, recip, dinv):
    blk = 400
    return pl.pallas_call(
        _final_body,
        grid=(N // blk,),
        in_specs=[
            pl.BlockSpec((blk, D), lambda i: (i, 0)),
            pl.BlockSpec((blk, D), lambda i: (i, 0)),
            pl.BlockSpec((1, D), lambda i: (0, 0)),
            pl.BlockSpec((blk, 1), lambda i: (i, 0)),
            pl.BlockSpec((blk, 1), lambda i: (i, 0)),
        ],
        out_specs=pl.BlockSpec((blk, D), lambda i: (i, 0)),
        out_shape=jax.ShapeDtypeStruct((N, D), jnp.float32),
    )(aggr, h, root_emb, recip.reshape(N, 1), dinv.reshape(N, 1))


@jax.jit
def kernel(x, edge_index, edge_attr, W_lin, b_lin, W_edge, b_edge, root_emb):
    row = edge_index[0]
    col = edge_index[1]

    deg_parts = _deg(row)
    deg = jnp.sum(deg_parts.reshape(NW, N), axis=0) + 1.0
    dinv = deg ** -0.5
    recip = 1.0 / deg

    h = _linear(x, W_lin, b_lin, 400)
    e = _linear(edge_attr, W_edge, b_edge, 6400)

    parts = _main(h, e, row, col, dinv)
    aggr = jnp.concatenate([parts[0, :HALF], parts[1, :HALF]], axis=0)
    return _final(aggr, h, root_emb, recip, dinv)
